# baseline (speedup 1.0000x reference)
import functools

import jax
import jax.numpy as jnp
from jax import lax
from jax.experimental import pallas as pl
from jax.experimental.pallas import tpu as pltpu
from jax.experimental.pallas import tpu_sc as plsc

NPTS = 10000
NPAD = 10240
NSAMP = 16
NPAIR = NPTS * NSAMP
BPAIR = NPAD * NSAMP
CH = 128
WD = 16
EPS = 1e-5
INF = float("inf")

PBLK = 512
RBLK = PBLK * NSAMP
NBLK = NPAD // PBLK
KROWS = 256
KGRID = NPAD // KROWS


def _proj_body(x_ref, wq_ref, bq_ref, wk_ref, bk_ref, wv_ref, bv_ref,
               xq_ref, xk_ref, xv_ref):
    x = x_ref[...]
    xq_ref[...] = jnp.dot(x, wq_ref[...], preferred_element_type=jnp.float32) + bq_ref[...]
    xk_ref[...] = jnp.dot(x, wk_ref[...], preferred_element_type=jnp.float32) + bk_ref[...]
    xv_ref[...] = jnp.dot(x, wv_ref[...], preferred_element_type=jnp.float32) + bv_ref[...]


def _projections(x_pad, Wq, bq, Wk, bk, Wv, bv):
    full = pl.BlockSpec((CH, CH), lambda i: (0, 0))
    brow = pl.BlockSpec((1, CH), lambda i: (0, 0))
    blk = pl.BlockSpec((PBLK, CH), lambda i: (i, 0))
    out = jax.ShapeDtypeStruct((NPAD, CH), jnp.float32)
    return pl.pallas_call(
        _proj_body,
        grid=(NBLK,),
        in_specs=[blk, full, brow, full, brow, full, brow],
        out_specs=[blk, blk, blk],
        out_shape=[out, out, out],
    )(x_pad, Wq, bq.reshape(1, CH), Wk, bk.reshape(1, CH), Wv, bv.reshape(1, CH))


def _merge_tile(pr, sq_r, pt_ref, t, d16, c16):
    pc = pt_ref[:, pl.ds(t * KROWS, KROWS)]
    sq_c = jnp.sum(pc * pc, axis=0, keepdims=True)
    dot = jax.lax.dot_general(pr, pc, (((1,), (0,)), ((), ())),
                              preferred_element_type=jnp.float32)
    d2t = (sq_r + sq_c) - 2.0 * dot
    colt = t * KROWS + lax.broadcasted_iota(jnp.int32, (KROWS, KROWS), 1)
    H = KROWS // 2
    swap = d2t[:, H:] < d2t[:, :H]
    tmin = jnp.where(swap, d2t[:, H:], d2t[:, :H])
    tmax = jnp.where(swap, d2t[:, :H], d2t[:, H:])
    tcmin = jnp.where(swap, colt[:, H:], colt[:, :H])
    tcmax = jnp.where(swap, colt[:, :H], colt[:, H:])
    dmin = jnp.concatenate([d16, tmin], axis=1)
    dmax = jnp.concatenate([jnp.full((KROWS, NSAMP), INF, jnp.float32), tmax],
                           axis=1)
    cmin = jnp.concatenate([c16, tcmin], axis=1)
    cmax = jnp.concatenate([c16, tcmax], axis=1)
    ms, ams = [], []
    for _ in range(NSAMP):
        m = jnp.min(dmin, axis=1, keepdims=True)
        am = jnp.min(jnp.where(dmin == m, cmin, NPAD), axis=1, keepdims=True)
        ms.append(m)
        ams.append(am)
        msel = (cmin == am) & (dmin == m)
        dmin = jnp.where(msel, dmax, dmin)
        cmin = jnp.where(msel, cmax, cmin)
        dmax = jnp.where(msel, INF, dmax)
    return jnp.concatenate(ms, axis=1), jnp.concatenate(ams, axis=1)


def _knn_body(p_ref, pt_ref, lb_ref, idx_ref):
    s = pl.program_id(0)
    pr = p_ref[...]
    sq_r = jnp.sum(pr * pr, axis=1, keepdims=True)
    d16 = jnp.full((KROWS, NSAMP), INF, jnp.float32)
    c16 = jnp.zeros((KROWS, NSAMP), jnp.int32)
    d16, c16 = _merge_tile(pr, sq_r, pt_ref, s, d16, c16)

    def body(t, carry):
        d16, c16 = carry
        u = jnp.max(d16)
        take = (lb_ref[s, t] < u) & (t != s)
        return lax.cond(
            take,
            lambda c: _merge_tile(pr, sq_r, pt_ref, t, c[0], c[1]),
            lambda c: c,
            (d16, c16))

    d16, c16 = lax.fori_loop(0, KGRID, body, (d16, c16))
    idx_ref[...] = c16


def _knn(p_pad8, pT, lb):
    return pl.pallas_call(
        _knn_body,
        grid=(KGRID,),
        in_specs=[pl.BlockSpec((KROWS, 8), lambda i: (i, 0)),
                  pl.BlockSpec((8, NPAD), lambda i: (0, 0)),
                  pl.BlockSpec(memory_space=pltpu.SMEM)],
        out_specs=pl.BlockSpec((KROWS, NSAMP), lambda i: (i, 0)),
        out_shape=jax.ShapeDtypeStruct((NPAD, NSAMP), jnp.int32),
    )(p_pad8, pT, lb)


def _sc_gather(xk, xv, p16, idxf):
    info = plsc.get_sparse_core_info()
    nw = info.num_cores * info.num_subcores
    bpw = BPAIR // nw
    chunk = 128
    nch = bpw // chunk
    nc = info.num_cores
    mesh = plsc.VectorSubcoreMesh(core_axis_name="c", subcore_axis_name="s")

    @functools.partial(
        pl.kernel, mesh=mesh,
        out_type=(jax.ShapeDtypeStruct((BPAIR, CH), jnp.float32),
                  jax.ShapeDtypeStruct((BPAIR, CH), jnp.float32),
                  jax.ShapeDtypeStruct((BPAIR, CH), jnp.float32)),
        scratch_types=[pltpu.VMEM((chunk,), jnp.int32),
                       pltpu.VMEM((chunk, CH), jnp.float32),
                       pltpu.VMEM((chunk, CH), jnp.float32),
                       pltpu.VMEM((chunk, CH), jnp.float32),
                       pltpu.SemaphoreType.DMA,
                       pltpu.SemaphoreType.DMA,
                       pltpu.SemaphoreType.DMA],
    )
    def gk(xk_h, xv_h, p_h, idx_h, okg, ovg, opg, idx_c, kb, vb, pb, s1, s2, s3):
        wid = lax.axis_index("s") * nc + lax.axis_index("c")
        base = wid * bpw

        def body(c, carry):
            off = base + c * chunk
            pltpu.sync_copy(idx_h.at[pl.ds(off, chunk)], idx_c)
            c1 = pltpu.async_copy(xk_h.at[idx_c], kb, s1)
            c2 = pltpu.async_copy(xv_h.at[idx_c], vb, s2)
            c3 = pltpu.async_copy(p_h.at[idx_c], pb, s3)
            c1.wait()
            c2.wait()
            c3.wait()
            pltpu.sync_copy(kb, okg.at[pl.ds(off, chunk)])
            pltpu.sync_copy(vb, ovg.at[pl.ds(off, chunk)])
            pltpu.sync_copy(pb, opg.at[pl.ds(off, chunk)])
            return carry

        lax.fori_loop(0, nch, body, 0)

    return gk(xk, xv, p16, idxf)


def _valid_mask(pid):
    row = lax.broadcasted_iota(jnp.int32, (RBLK, 1), 0)
    return jnp.where(pid * RBLK + row < NPAIR, 1.0, 0.0).astype(jnp.float32)


def _acc_init(pid, refs):
    @pl.when(pid == 0)
    def _():
        for r in refs:
            r[...] = jnp.zeros_like(r)


def _bn_scale(s1_ref, s2_ref, g, beta):
    mu = s1_ref[0:1, :] / NPAIR
    var = s2_ref[0:1, :] / NPAIR - mu * mu
    inv = g / jnp.sqrt(var + EPS)
    return mu, inv, beta


def _p_r(pg, wp1_ref, bp1_ref, mu_h, inv_h, betap, wp2_ref, bp2_ref):
    h = jnp.dot(pg, wp1_ref[...], preferred_element_type=jnp.float32) + bp1_ref[...]
    hbn = jnp.maximum((h - mu_h) * inv_h + betap, 0.0)
    return jnp.dot(hbn, wp2_ref[...], preferred_element_type=jnp.float32) + bp2_ref[...]


def _w0(kg_ref, xq_ref, pr):
    xq = xq_ref[...]
    xqe = jnp.reshape(jnp.broadcast_to(xq[:, None, :], (PBLK, NSAMP, CH)),
                      (RBLK, CH))
    return (kg_ref[...] - xqe) + pr


def _hstat_body(pg_ref, wp1_ref, bp1_ref, s1_ref, s2_ref):
    pid = pl.program_id(0)
    _acc_init(pid, (s1_ref, s2_ref))
    pg = pg_ref[...]
    h = jnp.dot(pg, wp1_ref[...], preferred_element_type=jnp.float32) + bp1_ref[...]
    v = _valid_mask(pid)
    s1_ref[...] += jnp.broadcast_to(jnp.sum(h * v, axis=0, keepdims=True), (8, WD))
    s2_ref[...] += jnp.broadcast_to(jnp.sum(h * h * v, axis=0, keepdims=True), (8, WD))


def _hstats(pg, Wp1p, bp1p):
    stat = pl.BlockSpec((8, WD), lambda i: (0, 0))
    return pl.pallas_call(
        _hstat_body,
        grid=(NBLK,),
        in_specs=[pl.BlockSpec((RBLK, CH), lambda i: (i, 0)),
                  pl.BlockSpec((CH, WD), lambda i: (0, 0)),
                  pl.BlockSpec((1, WD), lambda i: (0, 0))],
        out_specs=[stat, stat],
        out_shape=[jax.ShapeDtypeStruct((8, WD), jnp.float32)] * 2,
    )(pg, Wp1p, bp1p)


def _w0stat_body(kg_ref, pg_ref, xq_ref, hs1_ref, hs2_ref, wp1_ref, bp1_ref,
                 gp_ref, bep_ref, wp2_ref, bp2_ref, s1_ref, s2_ref):
    pid = pl.program_id(0)
    _acc_init(pid, (s1_ref, s2_ref))
    mu_h, inv_h, betap = _bn_scale(hs1_ref, hs2_ref, gp_ref[...], bep_ref[...])
    pr = _p_r(pg_ref[...], wp1_ref, bp1_ref, mu_h, inv_h, betap, wp2_ref, bp2_ref)
    w0 = _w0(kg_ref, xq_ref, pr)
    v = _valid_mask(pid)
    s1_ref[...] += jnp.broadcast_to(jnp.sum(w0 * v, axis=0, keepdims=True), (8, CH))
    s2_ref[...] += jnp.broadcast_to(jnp.sum(w0 * w0 * v, axis=0, keepdims=True), (8, CH))


def _w0stats(kg, pg, xq, hs1, hs2, Wp1p, bp1p, gPp, betaPp, Wp2p, bp2r):
    stat = pl.BlockSpec((8, CH), lambda i: (0, 0))
    stat16 = pl.BlockSpec((8, WD), lambda i: (0, 0))
    return pl.pallas_call(
        _w0stat_body,
        grid=(NBLK,),
        in_specs=[pl.BlockSpec((RBLK, CH), lambda i: (i, 0)),
                  pl.BlockSpec((RBLK, CH), lambda i: (i, 0)),
                  pl.BlockSpec((PBLK, CH), lambda i: (i, 0)),
                  stat16, stat16,
                  pl.BlockSpec((CH, WD), lambda i: (0, 0)),
                  pl.BlockSpec((1, WD), lambda i: (0, 0)),
                  pl.BlockSpec((1, WD), lambda i: (0, 0)),
                  pl.BlockSpec((1, WD), lambda i: (0, 0)),
                  pl.BlockSpec((WD, CH), lambda i: (0, 0)),
                  pl.BlockSpec((1, CH), lambda i: (0, 0))],
        out_specs=[stat, stat],
        out_shape=[jax.ShapeDtypeStruct((8, CH), jnp.float32)] * 2,
    )(kg, pg, xq, hs1, hs2, Wp1p, bp1p, gPp, betaPp, Wp2p, bp2r)


def _w1_body(kg_ref, pg_ref, xq_ref, hs1_ref, hs2_ref, ws1_ref, ws2_ref,
             wp1_ref, bp1_ref, gp_ref, bep_ref, wp2_ref, bp2_ref,
             g1_ref, be1_ref, w1_ref, b1_ref, w1out_ref, s1_ref, s2_ref):
    pid = pl.program_id(0)
    _acc_init(pid, (s1_ref, s2_ref))
    mu_h, inv_h, betap = _bn_scale(hs1_ref, hs2_ref, gp_ref[...], bep_ref[...])
    pr = _p_r(pg_ref[...], wp1_ref, bp1_ref, mu_h, inv_h, betap, wp2_ref, bp2_ref)
    w0 = _w0(kg_ref, xq_ref, pr)
    mu1, inv1, beta1 = _bn_scale(ws1_ref, ws2_ref, g1_ref[...], be1_ref[...])
    w0bn = jnp.maximum((w0 - mu1) * inv1 + beta1, 0.0)
    w1 = jnp.dot(w0bn, w1_ref[...], preferred_element_type=jnp.float32) + b1_ref[...]
    w1out_ref[...] = w1
    v = _valid_mask(pid)
    s1_ref[...] += jnp.broadcast_to(jnp.sum(w1 * v, axis=0, keepdims=True), (8, WD))
    s2_ref[...] += jnp.broadcast_to(jnp.sum(w1 * w1 * v, axis=0, keepdims=True), (8, WD))


def _w1pass(kg, pg, xq, hs1, hs2, ws1, ws2, Wp1p, bp1p, gPp, betaPp, Wp2p,
            bp2r, g1r, beta1r, W1, b1r):
    stat = pl.BlockSpec((8, WD), lambda i: (0, 0))
    statc = pl.BlockSpec((8, CH), lambda i: (0, 0))
    return pl.pallas_call(
        _w1_body,
        grid=(NBLK,),
        in_specs=[pl.BlockSpec((RBLK, CH), lambda i: (i, 0)),
                  pl.BlockSpec((RBLK, CH), lambda i: (i, 0)),
                  pl.BlockSpec((PBLK, CH), lambda i: (i, 0)),
                  stat, stat, statc, statc,
                  pl.BlockSpec((CH, WD), lambda i: (0, 0)),
                  pl.BlockSpec((1, WD), lambda i: (0, 0)),
                  pl.BlockSpec((1, WD), lambda i: (0, 0)),
                  pl.BlockSpec((1, WD), lambda i: (0, 0)),
                  pl.BlockSpec((WD, CH), lambda i: (0, 0)),
                  pl.BlockSpec((1, CH), lambda i: (0, 0)),
                  pl.BlockSpec((1, CH), lambda i: (0, 0)),
                  pl.BlockSpec((1, CH), lambda i: (0, 0)),
                  pl.BlockSpec((CH, WD), lambda i: (0, 0)),
                  pl.BlockSpec((1, WD), lambda i: (0, 0))],
        out_specs=[pl.BlockSpec((RBLK, WD), lambda i: (i, 0)), stat, stat],
        out_shape=[jax.ShapeDtypeStruct((BPAIR, WD), jnp.float32),
                   jax.ShapeDtypeStruct((8, WD), jnp.float32),
                   jax.ShapeDtypeStruct((8, WD), jnp.float32)],
    )(kg, pg, xq, hs1, hs2, ws1, ws2, Wp1p, bp1p, gPp, betaPp, Wp2p, bp2r,
      g1r, beta1r, W1, b1r)


def _final_body(w1_ref, vg_ref, pg_ref, w1s1_ref, w1s2_ref, hs1_ref, hs2_ref,
                wp1_ref, bp1_ref, gp_ref, bep_ref, wp2_ref, bp2_ref,
                g2_ref, be2_ref, w2_ref, b2_ref, tt_ref, out_ref):
    mu_h, inv_h, betap = _bn_scale(hs1_ref, hs2_ref, gp_ref[...], bep_ref[...])
    pr = _p_r(pg_ref[...], wp1_ref, bp1_ref, mu_h, inv_h, betap, wp2_ref, bp2_ref)
    mu2, inv2, beta2 = _bn_scale(w1s1_ref, w1s2_ref, g2_ref[...], be2_ref[...])
    w1bn = jnp.maximum((w1_ref[...] - mu2) * inv2 + beta2, 0.0)
    w2 = jnp.dot(w1bn, w2_ref[...], preferred_element_type=jnp.float32) + b2_ref[...]
    w2g = jnp.reshape(w2, (PBLK, NSAMP, WD))
    m = jnp.max(w2g, axis=1, keepdims=True)
    e = jnp.exp(w2g - m)
    s = jnp.sum(e, axis=1, keepdims=True)
    wsoft = jnp.reshape(e / s, (RBLK, WD))
    wfull = jnp.dot(wsoft, tt_ref[...], preferred_element_type=jnp.float32)
    vv = vg_ref[...] + pr
    prod = jnp.reshape(vv * wfull, (PBLK, NSAMP, CH))
    out_ref[...] = jnp.sum(prod, axis=1)


def _finalpass(w1, vg, pg, w1s1, w1s2, hs1, hs2, Wp1p, bp1p, gPp, betaPp,
               Wp2p, bp2r, g2r, beta2r, W2p, b2r, Ttile):
    stat = pl.BlockSpec((8, WD), lambda i: (0, 0))
    return pl.pallas_call(
        _final_body,
        grid=(NBLK,),
        in_specs=[pl.BlockSpec((RBLK, WD), lambda i: (i, 0)),
                  pl.BlockSpec((RBLK, CH), lambda i: (i, 0)),
                  pl.BlockSpec((RBLK, CH), lambda i: (i, 0)),
                  stat, stat, stat, stat,
                  pl.BlockSpec((CH, WD), lambda i: (0, 0)),
                  pl.BlockSpec((1, WD), lambda i: (0, 0)),
                  pl.BlockSpec((1, WD), lambda i: (0, 0)),
                  pl.BlockSpec((1, WD), lambda i: (0, 0)),
                  pl.BlockSpec((WD, CH), lambda i: (0, 0)),
                  pl.BlockSpec((1, CH), lambda i: (0, 0)),
                  pl.BlockSpec((1, WD), lambda i: (0, 0)),
                  pl.BlockSpec((1, WD), lambda i: (0, 0)),
                  pl.BlockSpec((WD, WD), lambda i: (0, 0)),
                  pl.BlockSpec((1, WD), lambda i: (0, 0)),
                  pl.BlockSpec((WD, CH), lambda i: (0, 0))],
        out_specs=pl.BlockSpec((PBLK, CH), lambda i: (i, 0)),
        out_shape=jax.ShapeDtypeStruct((NPAD, CH), jnp.float32),
    )(w1, vg, pg, w1s1, w1s2, hs1, hs2, Wp1p, bp1p, gPp, betaPp, Wp2p, bp2r,
      g2r, beta2r, W2p, b2r, Ttile)


def kernel(p, x, o, Wq, bq, Wk, bk, Wv, bv, Wp1, bp1, gP, betaP, Wp2, bp2,
           g1, beta1, W1, b1, g2, beta2, W2, b2):
    f32 = jnp.float32
    p = p.astype(f32)

    ci = jnp.clip((p * 3.2).astype(jnp.int32), 0, 31)

    def _spread(v):
        v = (v | (v << 8)) & 0x100F
        v = (v | (v << 4)) & 0x10C3
        v = (v | (v << 2)) & 0x1249
        return v

    code = (_spread(ci[:, 0]) << 2) | (_spread(ci[:, 1]) << 1) | _spread(ci[:, 2])
    perm = jnp.argsort(code)
    inv = jnp.argsort(perm)
    p = p[perm]
    x = x[perm]

    pad_n = NPAD - NPTS
    coords = jnp.pad(p, ((0, pad_n), (0, 0)), constant_values=1.0e3)
    p_pad8 = jnp.pad(coords, ((0, 0), (0, 5)))
    pT = p_pad8.T
    p128 = jnp.pad(coords, ((0, 0), (0, CH - 3)))
    x_pad = jnp.pad(x, ((0, pad_n), (0, 0)))

    pt3 = coords.reshape(KGRID, KROWS, 3)
    lo = pt3.min(axis=1)
    hi = pt3.max(axis=1)
    gap = jnp.maximum(jnp.maximum(lo[None, :, :] - hi[:, None, :],
                                  lo[:, None, :] - hi[None, :, :]), 0.0)
    lb = jnp.sum(gap * gap, axis=-1)

    Wp1p = jnp.pad(Wp1, ((0, CH - 3), (0, WD - 3)))
    bp1p = jnp.pad(bp1, (0, WD - 3)).reshape(1, WD)
    gPp = jnp.pad(gP, (0, WD - 3)).reshape(1, WD)
    betaPp = jnp.pad(betaP, (0, WD - 3)).reshape(1, WD)
    Wp2p = jnp.pad(Wp2, ((0, WD - 3), (0, 0)))
    bp2r = bp2.reshape(1, CH)
    g1r = g1.reshape(1, CH)
    beta1r = beta1.reshape(1, CH)
    b1r = b1.reshape(1, WD)
    g2r = g2.reshape(1, WD)
    beta2r = beta2.reshape(1, WD)
    b2r = b2.reshape(1, WD)
    Ttile = jnp.tile(jnp.eye(WD, dtype=f32), (1, CH // WD))

    xq, xk, xv = _projections(x_pad, Wq, bq, Wk, bk, Wv, bv)
    idx = _knn(p_pad8, pT, lb)
    idxf = idx.reshape(BPAIR)
    kg, vg, pg = _sc_gather(xk, xv, p128, idxf)

    hs1, hs2 = _hstats(pg, Wp1p, bp1p)
    ws1, ws2 = _w0stats(kg, pg, xq, hs1, hs2, Wp1p, bp1p, gPp, betaPp, Wp2p, bp2r)
    w1, w1s1, w1s2 = _w1pass(kg, pg, xq, hs1, hs2, ws1, ws2, Wp1p, bp1p, gPp,
                             betaPp, Wp2p, bp2r, g1r, beta1r, W1, b1r)
    out = _finalpass(w1, vg, pg, w1s1, w1s2, hs1, hs2, Wp1p, bp1p, gPp, betaPp,
                     Wp2p, bp2r, g2r, beta2r, W2, b2r, Ttile)
    return out[:NPTS][inv]


# speedup vs baseline: 2.2607x; 2.2607x over previous
import functools

import jax
import jax.numpy as jnp
from jax import lax
from jax.experimental import pallas as pl
from jax.experimental.pallas import tpu as pltpu
from jax.experimental.pallas import tpu_sc as plsc

NPTS = 10000
NPAD = 10240
NSAMP = 16
NPAIR = NPTS * NSAMP
BPAIR = NPAD * NSAMP
CH = 128
WD = 16
EPS = 1e-5
INF = float("inf")

PBLK = 512
RBLK = PBLK * NSAMP
NBLK = NPAD // PBLK
KROWS = 512
KGRID = NPAD // KROWS


def _proj_body(x_ref, wq_ref, bq_ref, wk_ref, bk_ref, wv_ref, bv_ref,
               xq_ref, xk_ref, xv_ref):
    x = x_ref[...]
    xq_ref[...] = jnp.dot(x, wq_ref[...], preferred_element_type=jnp.float32) + bq_ref[...]
    xk_ref[...] = jnp.dot(x, wk_ref[...], preferred_element_type=jnp.float32) + bk_ref[...]
    xv_ref[...] = jnp.dot(x, wv_ref[...], preferred_element_type=jnp.float32) + bv_ref[...]


def _projections(x_pad, Wq, bq, Wk, bk, Wv, bv):
    full = pl.BlockSpec((CH, CH), lambda i: (0, 0))
    brow = pl.BlockSpec((1, CH), lambda i: (0, 0))
    blk = pl.BlockSpec((PBLK, CH), lambda i: (i, 0))
    out = jax.ShapeDtypeStruct((NPAD, CH), jnp.float32)
    return pl.pallas_call(
        _proj_body,
        grid=(NBLK,),
        in_specs=[blk, full, brow, full, brow, full, brow],
        out_specs=[blk, blk, blk],
        out_shape=[out, out, out],
    )(x_pad, Wq, bq.reshape(1, CH), Wk, bk.reshape(1, CH), Wv, bv.reshape(1, CH))


def _knn_body(p_ref, pt_ref, idx_ref):
    pr = p_ref[...]
    pc = pt_ref[...]
    sq_r = jnp.sum(pr * pr, axis=1, keepdims=True)
    sq_c = jnp.sum(pc * pc, axis=0, keepdims=True)
    dot = jax.lax.dot_general(pr, pc, (((1,), (0,)), ((), ())),
                              preferred_element_type=jnp.float32)
    d2 = (sq_r + sq_c) - 2.0 * dot
    col = lax.broadcasted_iota(jnp.int32, (KROWS, NPAD), 1)
    d2 = jnp.where(col < NPTS, d2, INF)
    ams = []
    for _ in range(NSAMP):
        m = jnp.min(d2, axis=1, keepdims=True)
        am = jnp.min(jnp.where(d2 == m, col, NPAD), axis=1,
                     keepdims=True)
        ams.append(am)
        d2 = jnp.where(col == am, INF, d2)
    idx_ref[...] = jnp.concatenate(ams, axis=1)


def _knn(p_pad8, pT):
    return pl.pallas_call(
        _knn_body,
        grid=(KGRID,),
        in_specs=[pl.BlockSpec((KROWS, 8), lambda i: (i, 0)),
                  pl.BlockSpec((8, NPAD), lambda i: (0, 0))],
        out_specs=pl.BlockSpec((KROWS, NSAMP), lambda i: (i, 0)),
        out_shape=jax.ShapeDtypeStruct((NPAD, NSAMP), jnp.int32),
    )(p_pad8, pT)


def _sc_gather(xk, xv, p16, idxf):
    info = plsc.get_sparse_core_info()
    nw = info.num_cores * info.num_subcores
    bpw = BPAIR // nw
    chunk = 128
    nch = bpw // chunk
    nc = info.num_cores
    mesh = plsc.VectorSubcoreMesh(core_axis_name="c", subcore_axis_name="s")

    @functools.partial(
        pl.kernel, mesh=mesh,
        out_type=(jax.ShapeDtypeStruct((BPAIR, CH), jnp.float32),
                  jax.ShapeDtypeStruct((BPAIR, CH), jnp.float32),
                  jax.ShapeDtypeStruct((BPAIR, CH), jnp.float32)),
        scratch_types=[pltpu.VMEM((chunk,), jnp.int32),
                       pltpu.VMEM((chunk, CH), jnp.float32),
                       pltpu.VMEM((chunk, CH), jnp.float32),
                       pltpu.VMEM((chunk, CH), jnp.float32),
                       pltpu.SemaphoreType.DMA,
                       pltpu.SemaphoreType.DMA,
                       pltpu.SemaphoreType.DMA],
    )
    def gk(xk_h, xv_h, p_h, idx_h, okg, ovg, opg, idx_c, kb, vb, pb, s1, s2, s3):
        wid = lax.axis_index("s") * nc + lax.axis_index("c")
        base = wid * bpw

        def body(c, carry):
            off = base + c * chunk
            pltpu.sync_copy(idx_h.at[pl.ds(off, chunk)], idx_c)
            c1 = pltpu.async_copy(xk_h.at[idx_c], kb, s1)
            c2 = pltpu.async_copy(xv_h.at[idx_c], vb, s2)
            c3 = pltpu.async_copy(p_h.at[idx_c], pb, s3)
            c1.wait()
            c2.wait()
            c3.wait()
            pltpu.sync_copy(kb, okg.at[pl.ds(off, chunk)])
            pltpu.sync_copy(vb, ovg.at[pl.ds(off, chunk)])
            pltpu.sync_copy(pb, opg.at[pl.ds(off, chunk)])
            return carry

        lax.fori_loop(0, nch, body, 0)

    return gk(xk, xv, p16, idxf)


def _valid_mask(pid):
    row = lax.broadcasted_iota(jnp.int32, (RBLK, 1), 0)
    return jnp.where(pid * RBLK + row < NPAIR, 1.0, 0.0).astype(jnp.float32)


def _acc_init(pid, refs):
    @pl.when(pid == 0)
    def _():
        for r in refs:
            r[...] = jnp.zeros_like(r)


def _bn_scale(s1_ref, s2_ref, g, beta):
    mu = s1_ref[0:1, :] / NPAIR
    var = s2_ref[0:1, :] / NPAIR - mu * mu
    inv = g / jnp.sqrt(var + EPS)
    return mu, inv, beta


def _p_r(pg, wp1_ref, bp1_ref, mu_h, inv_h, betap, wp2_ref, bp2_ref):
    h = jnp.dot(pg, wp1_ref[...], preferred_element_type=jnp.float32) + bp1_ref[...]
    hbn = jnp.maximum((h - mu_h) * inv_h + betap, 0.0)
    return jnp.dot(hbn, wp2_ref[...], preferred_element_type=jnp.float32) + bp2_ref[...]


def _w0(kg_ref, xq_ref, pr):
    xq = xq_ref[...]
    xqe = jnp.reshape(jnp.broadcast_to(xq[:, None, :], (PBLK, NSAMP, CH)),
                      (RBLK, CH))
    return (kg_ref[...] - xqe) + pr


def _hstat_body(pg_ref, wp1_ref, bp1_ref, s1_ref, s2_ref):
    pid = pl.program_id(0)
    _acc_init(pid, (s1_ref, s2_ref))
    pg = pg_ref[...]
    h = jnp.dot(pg, wp1_ref[...], preferred_element_type=jnp.float32) + bp1_ref[...]
    v = _valid_mask(pid)
    s1_ref[...] += jnp.broadcast_to(jnp.sum(h * v, axis=0, keepdims=True), (8, WD))
    s2_ref[...] += jnp.broadcast_to(jnp.sum(h * h * v, axis=0, keepdims=True), (8, WD))


def _hstats(pg, Wp1p, bp1p):
    stat = pl.BlockSpec((8, WD), lambda i: (0, 0))
    return pl.pallas_call(
        _hstat_body,
        grid=(NBLK,),
        in_specs=[pl.BlockSpec((RBLK, CH), lambda i: (i, 0)),
                  pl.BlockSpec((CH, WD), lambda i: (0, 0)),
                  pl.BlockSpec((1, WD), lambda i: (0, 0))],
        out_specs=[stat, stat],
        out_shape=[jax.ShapeDtypeStruct((8, WD), jnp.float32)] * 2,
    )(pg, Wp1p, bp1p)


def _w0stat_body(kg_ref, pg_ref, xq_ref, hs1_ref, hs2_ref, wp1_ref, bp1_ref,
                 gp_ref, bep_ref, wp2_ref, bp2_ref, s1_ref, s2_ref):
    pid = pl.program_id(0)
    _acc_init(pid, (s1_ref, s2_ref))
    mu_h, inv_h, betap = _bn_scale(hs1_ref, hs2_ref, gp_ref[...], bep_ref[...])
    pr = _p_r(pg_ref[...], wp1_ref, bp1_ref, mu_h, inv_h, betap, wp2_ref, bp2_ref)
    w0 = _w0(kg_ref, xq_ref, pr)
    v = _valid_mask(pid)
    s1_ref[...] += jnp.broadcast_to(jnp.sum(w0 * v, axis=0, keepdims=True), (8, CH))
    s2_ref[...] += jnp.broadcast_to(jnp.sum(w0 * w0 * v, axis=0, keepdims=True), (8, CH))


def _w0stats(kg, pg, xq, hs1, hs2, Wp1p, bp1p, gPp, betaPp, Wp2p, bp2r):
    stat = pl.BlockSpec((8, CH), lambda i: (0, 0))
    stat16 = pl.BlockSpec((8, WD), lambda i: (0, 0))
    return pl.pallas_call(
        _w0stat_body,
        grid=(NBLK,),
        in_specs=[pl.BlockSpec((RBLK, CH), lambda i: (i, 0)),
                  pl.BlockSpec((RBLK, CH), lambda i: (i, 0)),
                  pl.BlockSpec((PBLK, CH), lambda i: (i, 0)),
                  stat16, stat16,
                  pl.BlockSpec((CH, WD), lambda i: (0, 0)),
                  pl.BlockSpec((1, WD), lambda i: (0, 0)),
                  pl.BlockSpec((1, WD), lambda i: (0, 0)),
                  pl.BlockSpec((1, WD), lambda i: (0, 0)),
                  pl.BlockSpec((WD, CH), lambda i: (0, 0)),
                  pl.BlockSpec((1, CH), lambda i: (0, 0))],
        out_specs=[stat, stat],
        out_shape=[jax.ShapeDtypeStruct((8, CH), jnp.float32)] * 2,
    )(kg, pg, xq, hs1, hs2, Wp1p, bp1p, gPp, betaPp, Wp2p, bp2r)


def _w1_body(kg_ref, pg_ref, xq_ref, hs1_ref, hs2_ref, ws1_ref, ws2_ref,
             wp1_ref, bp1_ref, gp_ref, bep_ref, wp2_ref, bp2_ref,
             g1_ref, be1_ref, w1_ref, b1_ref, w1out_ref, s1_ref, s2_ref):
    pid = pl.program_id(0)
    _acc_init(pid, (s1_ref, s2_ref))
    mu_h, inv_h, betap = _bn_scale(hs1_ref, hs2_ref, gp_ref[...], bep_ref[...])
    pr = _p_r(pg_ref[...], wp1_ref, bp1_ref, mu_h, inv_h, betap, wp2_ref, bp2_ref)
    w0 = _w0(kg_ref, xq_ref, pr)
    mu1, inv1, beta1 = _bn_scale(ws1_ref, ws2_ref, g1_ref[...], be1_ref[...])
    w0bn = jnp.maximum((w0 - mu1) * inv1 + beta1, 0.0)
    w1 = jnp.dot(w0bn, w1_ref[...], preferred_element_type=jnp.float32) + b1_ref[...]
    w1out_ref[...] = w1
    v = _valid_mask(pid)
    s1_ref[...] += jnp.broadcast_to(jnp.sum(w1 * v, axis=0, keepdims=True), (8, WD))
    s2_ref[...] += jnp.broadcast_to(jnp.sum(w1 * w1 * v, axis=0, keepdims=True), (8, WD))


def _w1pass(kg, pg, xq, hs1, hs2, ws1, ws2, Wp1p, bp1p, gPp, betaPp, Wp2p,
            bp2r, g1r, beta1r, W1, b1r):
    stat = pl.BlockSpec((8, WD), lambda i: (0, 0))
    statc = pl.BlockSpec((8, CH), lambda i: (0, 0))
    return pl.pallas_call(
        _w1_body,
        grid=(NBLK,),
        in_specs=[pl.BlockSpec((RBLK, CH), lambda i: (i, 0)),
                  pl.BlockSpec((RBLK, CH), lambda i: (i, 0)),
                  pl.BlockSpec((PBLK, CH), lambda i: (i, 0)),
                  stat, stat, statc, statc,
                  pl.BlockSpec((CH, WD), lambda i: (0, 0)),
                  pl.BlockSpec((1, WD), lambda i: (0, 0)),
                  pl.BlockSpec((1, WD), lambda i: (0, 0)),
                  pl.BlockSpec((1, WD), lambda i: (0, 0)),
                  pl.BlockSpec((WD, CH), lambda i: (0, 0)),
                  pl.BlockSpec((1, CH), lambda i: (0, 0)),
                  pl.BlockSpec((1, CH), lambda i: (0, 0)),
                  pl.BlockSpec((1, CH), lambda i: (0, 0)),
                  pl.BlockSpec((CH, WD), lambda i: (0, 0)),
                  pl.BlockSpec((1, WD), lambda i: (0, 0))],
        out_specs=[pl.BlockSpec((RBLK, WD), lambda i: (i, 0)), stat, stat],
        out_shape=[jax.ShapeDtypeStruct((BPAIR, WD), jnp.float32),
                   jax.ShapeDtypeStruct((8, WD), jnp.float32),
                   jax.ShapeDtypeStruct((8, WD), jnp.float32)],
    )(kg, pg, xq, hs1, hs2, ws1, ws2, Wp1p, bp1p, gPp, betaPp, Wp2p, bp2r,
      g1r, beta1r, W1, b1r)


def _final_body(w1_ref, vg_ref, pg_ref, w1s1_ref, w1s2_ref, hs1_ref, hs2_ref,
                wp1_ref, bp1_ref, gp_ref, bep_ref, wp2_ref, bp2_ref,
                g2_ref, be2_ref, w2_ref, b2_ref, tt_ref, out_ref):
    mu_h, inv_h, betap = _bn_scale(hs1_ref, hs2_ref, gp_ref[...], bep_ref[...])
    pr = _p_r(pg_ref[...], wp1_ref, bp1_ref, mu_h, inv_h, betap, wp2_ref, bp2_ref)
    mu2, inv2, beta2 = _bn_scale(w1s1_ref, w1s2_ref, g2_ref[...], be2_ref[...])
    w1bn = jnp.maximum((w1_ref[...] - mu2) * inv2 + beta2, 0.0)
    w2 = jnp.dot(w1bn, w2_ref[...], preferred_element_type=jnp.float32) + b2_ref[...]
    w2g = jnp.reshape(w2, (PBLK, NSAMP, WD))
    m = jnp.max(w2g, axis=1, keepdims=True)
    e = jnp.exp(w2g - m)
    s = jnp.sum(e, axis=1, keepdims=True)
    wsoft = jnp.reshape(e / s, (RBLK, WD))
    wfull = jnp.dot(wsoft, tt_ref[...], preferred_element_type=jnp.float32)
    vv = vg_ref[...] + pr
    prod = jnp.reshape(vv * wfull, (PBLK, NSAMP, CH))
    out_ref[...] = jnp.sum(prod, axis=1)


def _finalpass(w1, vg, pg, w1s1, w1s2, hs1, hs2, Wp1p, bp1p, gPp, betaPp,
               Wp2p, bp2r, g2r, beta2r, W2p, b2r, Ttile):
    stat = pl.BlockSpec((8, WD), lambda i: (0, 0))
    return pl.pallas_call(
        _final_body,
        grid=(NBLK,),
        in_specs=[pl.BlockSpec((RBLK, WD), lambda i: (i, 0)),
                  pl.BlockSpec((RBLK, CH), lambda i: (i, 0)),
                  pl.BlockSpec((RBLK, CH), lambda i: (i, 0)),
                  stat, stat, stat, stat,
                  pl.BlockSpec((CH, WD), lambda i: (0, 0)),
                  pl.BlockSpec((1, WD), lambda i: (0, 0)),
                  pl.BlockSpec((1, WD), lambda i: (0, 0)),
                  pl.BlockSpec((1, WD), lambda i: (0, 0)),
                  pl.BlockSpec((WD, CH), lambda i: (0, 0)),
                  pl.BlockSpec((1, CH), lambda i: (0, 0)),
                  pl.BlockSpec((1, WD), lambda i: (0, 0)),
                  pl.BlockSpec((1, WD), lambda i: (0, 0)),
                  pl.BlockSpec((WD, WD), lambda i: (0, 0)),
                  pl.BlockSpec((1, WD), lambda i: (0, 0)),
                  pl.BlockSpec((WD, CH), lambda i: (0, 0))],
        out_specs=pl.BlockSpec((PBLK, CH), lambda i: (i, 0)),
        out_shape=jax.ShapeDtypeStruct((NPAD, CH), jnp.float32),
    )(w1, vg, pg, w1s1, w1s2, hs1, hs2, Wp1p, bp1p, gPp, betaPp, Wp2p, bp2r,
      g2r, beta2r, W2p, b2r, Ttile)


def kernel(p, x, o, Wq, bq, Wk, bk, Wv, bv, Wp1, bp1, gP, betaP, Wp2, bp2,
           g1, beta1, W1, b1, g2, beta2, W2, b2):
    f32 = jnp.float32
    p = p.astype(f32)
    pad_n = NPAD - NPTS
    p_pad8 = jnp.pad(p, ((0, pad_n), (0, 5)))
    pT = p_pad8.T
    p128 = jnp.pad(p, ((0, pad_n), (0, CH - 3)))
    x_pad = jnp.pad(x, ((0, pad_n), (0, 0)))

    Wp1p = jnp.pad(Wp1, ((0, CH - 3), (0, WD - 3)))
    bp1p = jnp.pad(bp1, (0, WD - 3)).reshape(1, WD)
    gPp = jnp.pad(gP, (0, WD - 3)).reshape(1, WD)
    betaPp = jnp.pad(betaP, (0, WD - 3)).reshape(1, WD)
    Wp2p = jnp.pad(Wp2, ((0, WD - 3), (0, 0)))
    bp2r = bp2.reshape(1, CH)
    g1r = g1.reshape(1, CH)
    beta1r = beta1.reshape(1, CH)
    b1r = b1.reshape(1, WD)
    g2r = g2.reshape(1, WD)
    beta2r = beta2.reshape(1, WD)
    b2r = b2.reshape(1, WD)
    Ttile = jnp.tile(jnp.eye(WD, dtype=f32), (1, CH // WD))

    xq, xk, xv = _projections(x_pad, Wq, bq, Wk, bk, Wv, bv)
    idx = _knn(p_pad8, pT)
    idxf = idx.reshape(BPAIR)
    kg, vg, pg = _sc_gather(xk, xv, p128, idxf)

    hs1, hs2 = _hstats(pg, Wp1p, bp1p)
    ws1, ws2 = _w0stats(kg, pg, xq, hs1, hs2, Wp1p, bp1p, gPp, betaPp, Wp2p, bp2r)
    w1, w1s1, w1s2 = _w1pass(kg, pg, xq, hs1, hs2, ws1, ws2, Wp1p, bp1p, gPp,
                             betaPp, Wp2p, bp2r, g1r, beta1r, W1, b1r)
    out = _finalpass(w1, vg, pg, w1s1, w1s2, hs1, hs2, Wp1p, bp1p, gPp, betaPp,
                     Wp2p, bp2r, g2r, beta2r, W2, b2r, Ttile)
    return out[:NPTS]


# speedup vs baseline: 2.3031x; 1.0188x over previous
import functools

import jax
import jax.numpy as jnp
from jax import lax
from jax.experimental import pallas as pl
from jax.experimental.pallas import tpu as pltpu
from jax.experimental.pallas import tpu_sc as plsc

NPTS = 10000
NPAD = 10240
NSAMP = 16
NPAIR = NPTS * NSAMP
BPAIR = NPAD * NSAMP
CH = 128
WD = 16
EPS = 1e-5
INF = float("inf")

PBLK = 512
RBLK = PBLK * NSAMP
NBLK = NPAD // PBLK
KROWS = 1024
KGRID = NPAD // KROWS


def _proj_body(x_ref, wq_ref, bq_ref, wk_ref, bk_ref, wv_ref, bv_ref,
               xq_ref, xk_ref, xv_ref):
    x = x_ref[...]
    xq_ref[...] = jnp.dot(x, wq_ref[...], preferred_element_type=jnp.float32) + bq_ref[...]
    xk_ref[...] = jnp.dot(x, wk_ref[...], preferred_element_type=jnp.float32) + bk_ref[...]
    xv_ref[...] = jnp.dot(x, wv_ref[...], preferred_element_type=jnp.float32) + bv_ref[...]


def _projections(x_pad, Wq, bq, Wk, bk, Wv, bv):
    full = pl.BlockSpec((CH, CH), lambda i: (0, 0))
    brow = pl.BlockSpec((1, CH), lambda i: (0, 0))
    blk = pl.BlockSpec((PBLK, CH), lambda i: (i, 0))
    out = jax.ShapeDtypeStruct((NPAD, CH), jnp.float32)
    return pl.pallas_call(
        _proj_body,
        grid=(NBLK,),
        in_specs=[blk, full, brow, full, brow, full, brow],
        out_specs=[blk, blk, blk],
        out_shape=[out, out, out],
    )(x_pad, Wq, bq.reshape(1, CH), Wk, bk.reshape(1, CH), Wv, bv.reshape(1, CH))


def _knn_body(p_ref, pt_ref, idx_ref):
    pr = p_ref[...]
    pc = pt_ref[...]
    sq_r = jnp.sum(pr * pr, axis=1, keepdims=True)
    sq_c = jnp.sum(pc * pc, axis=0, keepdims=True)
    dot = jax.lax.dot_general(pr, pc, (((1,), (0,)), ((), ())),
                              preferred_element_type=jnp.float32)
    d2 = (sq_r + sq_c) - 2.0 * dot
    col = lax.broadcasted_iota(jnp.int32, (KROWS, NPAD), 1)
    d2 = jnp.where(col < NPTS, d2, INF)
    ams = []
    for _ in range(NSAMP):
        m = jnp.min(d2, axis=1, keepdims=True)
        am = jnp.min(jnp.where(d2 == m, col, NPAD), axis=1,
                     keepdims=True)
        ams.append(am)
        d2 = jnp.where(col == am, INF, d2)
    idx_ref[...] = jnp.concatenate(ams, axis=1)


def _knn(p_pad8, pT):
    return pl.pallas_call(
        _knn_body,
        grid=(KGRID,),
        in_specs=[pl.BlockSpec((KROWS, 8), lambda i: (i, 0)),
                  pl.BlockSpec((8, NPAD), lambda i: (0, 0))],
        out_specs=pl.BlockSpec((KROWS, NSAMP), lambda i: (i, 0)),
        out_shape=jax.ShapeDtypeStruct((NPAD, NSAMP), jnp.int32),
    )(p_pad8, pT)


def _sc_gather(xk, xv, p16, idxf):
    info = plsc.get_sparse_core_info()
    nw = info.num_cores * info.num_subcores
    bpw = BPAIR // nw
    chunk = 128
    nch = bpw // chunk
    nc = info.num_cores
    mesh = plsc.VectorSubcoreMesh(core_axis_name="c", subcore_axis_name="s")

    @functools.partial(
        pl.kernel, mesh=mesh,
        out_type=(jax.ShapeDtypeStruct((BPAIR, CH), jnp.float32),
                  jax.ShapeDtypeStruct((BPAIR, CH), jnp.float32),
                  jax.ShapeDtypeStruct((BPAIR, CH), jnp.float32)),
        scratch_types=[pltpu.VMEM((chunk,), jnp.int32),
                       pltpu.VMEM((chunk, CH), jnp.float32),
                       pltpu.VMEM((chunk, CH), jnp.float32),
                       pltpu.VMEM((chunk, CH), jnp.float32),
                       pltpu.SemaphoreType.DMA,
                       pltpu.SemaphoreType.DMA,
                       pltpu.SemaphoreType.DMA],
    )
    def gk(xk_h, xv_h, p_h, idx_h, okg, ovg, opg, idx_c, kb, vb, pb, s1, s2, s3):
        wid = lax.axis_index("s") * nc + lax.axis_index("c")
        base = wid * bpw

        def body(c, carry):
            off = base + c * chunk
            pltpu.sync_copy(idx_h.at[pl.ds(off, chunk)], idx_c)
            c1 = pltpu.async_copy(xk_h.at[idx_c], kb, s1)
            c2 = pltpu.async_copy(xv_h.at[idx_c], vb, s2)
            c3 = pltpu.async_copy(p_h.at[idx_c], pb, s3)
            c1.wait()
            c2.wait()
            c3.wait()
            pltpu.sync_copy(kb, okg.at[pl.ds(off, chunk)])
            pltpu.sync_copy(vb, ovg.at[pl.ds(off, chunk)])
            pltpu.sync_copy(pb, opg.at[pl.ds(off, chunk)])
            return carry

        lax.fori_loop(0, nch, body, 0)

    return gk(xk, xv, p16, idxf)


def _valid_mask(pid):
    row = lax.broadcasted_iota(jnp.int32, (RBLK, 1), 0)
    return jnp.where(pid * RBLK + row < NPAIR, 1.0, 0.0).astype(jnp.float32)


def _acc_init(pid, refs):
    @pl.when(pid == 0)
    def _():
        for r in refs:
            r[...] = jnp.zeros_like(r)


def _bn_scale(s1_ref, s2_ref, g, beta):
    mu = s1_ref[0:1, :] / NPAIR
    var = s2_ref[0:1, :] / NPAIR - mu * mu
    inv = g / jnp.sqrt(var + EPS)
    return mu, inv, beta


def _p_r(pg, wp1_ref, bp1_ref, mu_h, inv_h, betap, wp2_ref, bp2_ref):
    h = jnp.dot(pg, wp1_ref[...], preferred_element_type=jnp.float32) + bp1_ref[...]
    hbn = jnp.maximum((h - mu_h) * inv_h + betap, 0.0)
    return jnp.dot(hbn, wp2_ref[...], preferred_element_type=jnp.float32) + bp2_ref[...]


def _w0(kg_ref, xq_ref, pr):
    xq = xq_ref[...]
    xqe = jnp.reshape(jnp.broadcast_to(xq[:, None, :], (PBLK, NSAMP, CH)),
                      (RBLK, CH))
    return (kg_ref[...] - xqe) + pr


def _hstat_body(pg_ref, wp1_ref, bp1_ref, s1_ref, s2_ref):
    pid = pl.program_id(0)
    _acc_init(pid, (s1_ref, s2_ref))
    pg = pg_ref[...]
    h = jnp.dot(pg, wp1_ref[...], preferred_element_type=jnp.float32) + bp1_ref[...]
    v = _valid_mask(pid)
    s1_ref[...] += jnp.broadcast_to(jnp.sum(h * v, axis=0, keepdims=True), (8, WD))
    s2_ref[...] += jnp.broadcast_to(jnp.sum(h * h * v, axis=0, keepdims=True), (8, WD))


def _hstats(pg, Wp1p, bp1p):
    stat = pl.BlockSpec((8, WD), lambda i: (0, 0))
    return pl.pallas_call(
        _hstat_body,
        grid=(NBLK,),
        in_specs=[pl.BlockSpec((RBLK, CH), lambda i: (i, 0)),
                  pl.BlockSpec((CH, WD), lambda i: (0, 0)),
                  pl.BlockSpec((1, WD), lambda i: (0, 0))],
        out_specs=[stat, stat],
        out_shape=[jax.ShapeDtypeStruct((8, WD), jnp.float32)] * 2,
    )(pg, Wp1p, bp1p)


def _w0stat_body(kg_ref, pg_ref, xq_ref, hs1_ref, hs2_ref, wp1_ref, bp1_ref,
                 gp_ref, bep_ref, wp2_ref, bp2_ref, s1_ref, s2_ref):
    pid = pl.program_id(0)
    _acc_init(pid, (s1_ref, s2_ref))
    mu_h, inv_h, betap = _bn_scale(hs1_ref, hs2_ref, gp_ref[...], bep_ref[...])
    pr = _p_r(pg_ref[...], wp1_ref, bp1_ref, mu_h, inv_h, betap, wp2_ref, bp2_ref)
    w0 = _w0(kg_ref, xq_ref, pr)
    v = _valid_mask(pid)
    s1_ref[...] += jnp.broadcast_to(jnp.sum(w0 * v, axis=0, keepdims=True), (8, CH))
    s2_ref[...] += jnp.broadcast_to(jnp.sum(w0 * w0 * v, axis=0, keepdims=True), (8, CH))


def _w0stats(kg, pg, xq, hs1, hs2, Wp1p, bp1p, gPp, betaPp, Wp2p, bp2r):
    stat = pl.BlockSpec((8, CH), lambda i: (0, 0))
    stat16 = pl.BlockSpec((8, WD), lambda i: (0, 0))
    return pl.pallas_call(
        _w0stat_body,
        grid=(NBLK,),
        in_specs=[pl.BlockSpec((RBLK, CH), lambda i: (i, 0)),
                  pl.BlockSpec((RBLK, CH), lambda i: (i, 0)),
                  pl.BlockSpec((PBLK, CH), lambda i: (i, 0)),
                  stat16, stat16,
                  pl.BlockSpec((CH, WD), lambda i: (0, 0)),
                  pl.BlockSpec((1, WD), lambda i: (0, 0)),
                  pl.BlockSpec((1, WD), lambda i: (0, 0)),
                  pl.BlockSpec((1, WD), lambda i: (0, 0)),
                  pl.BlockSpec((WD, CH), lambda i: (0, 0)),
                  pl.BlockSpec((1, CH), lambda i: (0, 0))],
        out_specs=[stat, stat],
        out_shape=[jax.ShapeDtypeStruct((8, CH), jnp.float32)] * 2,
    )(kg, pg, xq, hs1, hs2, Wp1p, bp1p, gPp, betaPp, Wp2p, bp2r)


def _w1_body(kg_ref, pg_ref, xq_ref, hs1_ref, hs2_ref, ws1_ref, ws2_ref,
             wp1_ref, bp1_ref, gp_ref, bep_ref, wp2_ref, bp2_ref,
             g1_ref, be1_ref, w1_ref, b1_ref, w1out_ref, s1_ref, s2_ref):
    pid = pl.program_id(0)
    _acc_init(pid, (s1_ref, s2_ref))
    mu_h, inv_h, betap = _bn_scale(hs1_ref, hs2_ref, gp_ref[...], bep_ref[...])
    pr = _p_r(pg_ref[...], wp1_ref, bp1_ref, mu_h, inv_h, betap, wp2_ref, bp2_ref)
    w0 = _w0(kg_ref, xq_ref, pr)
    mu1, inv1, beta1 = _bn_scale(ws1_ref, ws2_ref, g1_ref[...], be1_ref[...])
    w0bn = jnp.maximum((w0 - mu1) * inv1 + beta1, 0.0)
    w1 = jnp.dot(w0bn, w1_ref[...], preferred_element_type=jnp.float32) + b1_ref[...]
    w1out_ref[...] = w1
    v = _valid_mask(pid)
    s1_ref[...] += jnp.broadcast_to(jnp.sum(w1 * v, axis=0, keepdims=True), (8, WD))
    s2_ref[...] += jnp.broadcast_to(jnp.sum(w1 * w1 * v, axis=0, keepdims=True), (8, WD))


def _w1pass(kg, pg, xq, hs1, hs2, ws1, ws2, Wp1p, bp1p, gPp, betaPp, Wp2p,
            bp2r, g1r, beta1r, W1, b1r):
    stat = pl.BlockSpec((8, WD), lambda i: (0, 0))
    statc = pl.BlockSpec((8, CH), lambda i: (0, 0))
    return pl.pallas_call(
        _w1_body,
        grid=(NBLK,),
        in_specs=[pl.BlockSpec((RBLK, CH), lambda i: (i, 0)),
                  pl.BlockSpec((RBLK, CH), lambda i: (i, 0)),
                  pl.BlockSpec((PBLK, CH), lambda i: (i, 0)),
                  stat, stat, statc, statc,
                  pl.BlockSpec((CH, WD), lambda i: (0, 0)),
                  pl.BlockSpec((1, WD), lambda i: (0, 0)),
                  pl.BlockSpec((1, WD), lambda i: (0, 0)),
                  pl.BlockSpec((1, WD), lambda i: (0, 0)),
                  pl.BlockSpec((WD, CH), lambda i: (0, 0)),
                  pl.BlockSpec((1, CH), lambda i: (0, 0)),
                  pl.BlockSpec((1, CH), lambda i: (0, 0)),
                  pl.BlockSpec((1, CH), lambda i: (0, 0)),
                  pl.BlockSpec((CH, WD), lambda i: (0, 0)),
                  pl.BlockSpec((1, WD), lambda i: (0, 0))],
        out_specs=[pl.BlockSpec((RBLK, WD), lambda i: (i, 0)), stat, stat],
        out_shape=[jax.ShapeDtypeStruct((BPAIR, WD), jnp.float32),
                   jax.ShapeDtypeStruct((8, WD), jnp.float32),
                   jax.ShapeDtypeStruct((8, WD), jnp.float32)],
    )(kg, pg, xq, hs1, hs2, ws1, ws2, Wp1p, bp1p, gPp, betaPp, Wp2p, bp2r,
      g1r, beta1r, W1, b1r)


def _final_body(w1_ref, vg_ref, pg_ref, w1s1_ref, w1s2_ref, hs1_ref, hs2_ref,
                wp1_ref, bp1_ref, gp_ref, bep_ref, wp2_ref, bp2_ref,
                g2_ref, be2_ref, w2_ref, b2_ref, tt_ref, out_ref):
    mu_h, inv_h, betap = _bn_scale(hs1_ref, hs2_ref, gp_ref[...], bep_ref[...])
    pr = _p_r(pg_ref[...], wp1_ref, bp1_ref, mu_h, inv_h, betap, wp2_ref, bp2_ref)
    mu2, inv2, beta2 = _bn_scale(w1s1_ref, w1s2_ref, g2_ref[...], be2_ref[...])
    w1bn = jnp.maximum((w1_ref[...] - mu2) * inv2 + beta2, 0.0)
    w2 = jnp.dot(w1bn, w2_ref[...], preferred_element_type=jnp.float32) + b2_ref[...]
    w2g = jnp.reshape(w2, (PBLK, NSAMP, WD))
    m = jnp.max(w2g, axis=1, keepdims=True)
    e = jnp.exp(w2g - m)
    s = jnp.sum(e, axis=1, keepdims=True)
    wsoft = jnp.reshape(e / s, (RBLK, WD))
    wfull = jnp.dot(wsoft, tt_ref[...], preferred_element_type=jnp.float32)
    vv = vg_ref[...] + pr
    prod = jnp.reshape(vv * wfull, (PBLK, NSAMP, CH))
    out_ref[...] = jnp.sum(prod, axis=1)


def _finalpass(w1, vg, pg, w1s1, w1s2, hs1, hs2, Wp1p, bp1p, gPp, betaPp,
               Wp2p, bp2r, g2r, beta2r, W2p, b2r, Ttile):
    stat = pl.BlockSpec((8, WD), lambda i: (0, 0))
    return pl.pallas_call(
        _final_body,
        grid=(NBLK,),
        in_specs=[pl.BlockSpec((RBLK, WD), lambda i: (i, 0)),
                  pl.BlockSpec((RBLK, CH), lambda i: (i, 0)),
                  pl.BlockSpec((RBLK, CH), lambda i: (i, 0)),
                  stat, stat, stat, stat,
                  pl.BlockSpec((CH, WD), lambda i: (0, 0)),
                  pl.BlockSpec((1, WD), lambda i: (0, 0)),
                  pl.BlockSpec((1, WD), lambda i: (0, 0)),
                  pl.BlockSpec((1, WD), lambda i: (0, 0)),
                  pl.BlockSpec((WD, CH), lambda i: (0, 0)),
                  pl.BlockSpec((1, CH), lambda i: (0, 0)),
                  pl.BlockSpec((1, WD), lambda i: (0, 0)),
                  pl.BlockSpec((1, WD), lambda i: (0, 0)),
                  pl.BlockSpec((WD, WD), lambda i: (0, 0)),
                  pl.BlockSpec((1, WD), lambda i: (0, 0)),
                  pl.BlockSpec((WD, CH), lambda i: (0, 0))],
        out_specs=pl.BlockSpec((PBLK, CH), lambda i: (i, 0)),
        out_shape=jax.ShapeDtypeStruct((NPAD, CH), jnp.float32),
    )(w1, vg, pg, w1s1, w1s2, hs1, hs2, Wp1p, bp1p, gPp, betaPp, Wp2p, bp2r,
      g2r, beta2r, W2p, b2r, Ttile)


def kernel(p, x, o, Wq, bq, Wk, bk, Wv, bv, Wp1, bp1, gP, betaP, Wp2, bp2,
           g1, beta1, W1, b1, g2, beta2, W2, b2):
    f32 = jnp.float32
    p = p.astype(f32)
    pad_n = NPAD - NPTS
    p_pad8 = jnp.pad(p, ((0, pad_n), (0, 5)))
    pT = p_pad8.T
    p128 = jnp.pad(p, ((0, pad_n), (0, CH - 3)))
    x_pad = jnp.pad(x, ((0, pad_n), (0, 0)))

    Wp1p = jnp.pad(Wp1, ((0, CH - 3), (0, WD - 3)))
    bp1p = jnp.pad(bp1, (0, WD - 3)).reshape(1, WD)
    gPp = jnp.pad(gP, (0, WD - 3)).reshape(1, WD)
    betaPp = jnp.pad(betaP, (0, WD - 3)).reshape(1, WD)
    Wp2p = jnp.pad(Wp2, ((0, WD - 3), (0, 0)))
    bp2r = bp2.reshape(1, CH)
    g1r = g1.reshape(1, CH)
    beta1r = beta1.reshape(1, CH)
    b1r = b1.reshape(1, WD)
    g2r = g2.reshape(1, WD)
    beta2r = beta2.reshape(1, WD)
    b2r = b2.reshape(1, WD)
    Ttile = jnp.tile(jnp.eye(WD, dtype=f32), (1, CH // WD))

    xq, xk, xv = _projections(x_pad, Wq, bq, Wk, bk, Wv, bv)
    idx = _knn(p_pad8, pT)
    idxf = idx.reshape(BPAIR)
    kg, vg, pg = _sc_gather(xk, xv, p128, idxf)

    hs1, hs2 = _hstats(pg, Wp1p, bp1p)
    ws1, ws2 = _w0stats(kg, pg, xq, hs1, hs2, Wp1p, bp1p, gPp, betaPp, Wp2p, bp2r)
    w1, w1s1, w1s2 = _w1pass(kg, pg, xq, hs1, hs2, ws1, ws2, Wp1p, bp1p, gPp,
                             betaPp, Wp2p, bp2r, g1r, beta1r, W1, b1r)
    out = _finalpass(w1, vg, pg, w1s1, w1s2, hs1, hs2, Wp1p, bp1p, gPp, betaPp,
                     Wp2p, bp2r, g2r, beta2r, W2, b2r, Ttile)
    return out[:NPTS]


# speedup vs baseline: 2.4641x; 1.0699x over previous
import functools

import jax
import jax.numpy as jnp
from jax import lax
from jax.experimental import pallas as pl
from jax.experimental.pallas import tpu as pltpu
from jax.experimental.pallas import tpu_sc as plsc

NPTS = 10000
NPAD = 10240
NSAMP = 16
NPAIR = NPTS * NSAMP
BPAIR = NPAD * NSAMP
CH = 128
WD = 16
EPS = 1e-5
INF = float("inf")

PBLK = 512
RBLK = PBLK * NSAMP
NBLK = NPAD // PBLK
KROWS = 1024
KGRID = NPAD // KROWS


def _proj_body(x_ref, wq_ref, bq_ref, wk_ref, bk_ref, wv_ref, bv_ref,
               xq_ref, xk_ref, xv_ref):
    x = x_ref[...]
    xq_ref[...] = jnp.dot(x, wq_ref[...], preferred_element_type=jnp.float32) + bq_ref[...]
    xk_ref[...] = jnp.dot(x, wk_ref[...], preferred_element_type=jnp.float32) + bk_ref[...]
    xv_ref[...] = jnp.dot(x, wv_ref[...], preferred_element_type=jnp.float32) + bv_ref[...]


def _projections(x_pad, Wq, bq, Wk, bk, Wv, bv):
    full = pl.BlockSpec((CH, CH), lambda i: (0, 0))
    brow = pl.BlockSpec((1, CH), lambda i: (0, 0))
    blk = pl.BlockSpec((PBLK, CH), lambda i: (i, 0))
    out = jax.ShapeDtypeStruct((NPAD, CH), jnp.float32)
    return pl.pallas_call(
        _proj_body,
        grid=(NBLK,),
        in_specs=[blk, full, brow, full, brow, full, brow],
        out_specs=[blk, blk, blk],
        out_shape=[out, out, out],
    )(x_pad, Wq, bq.reshape(1, CH), Wk, bk.reshape(1, CH), Wv, bv.reshape(1, CH))


def _knn_body(p_ref, pt_ref, idx_ref):
    pr = p_ref[...]
    pc = pt_ref[...]
    sq_r = jnp.sum(pr * pr, axis=1, keepdims=True)
    sq_c = jnp.sum(pc * pc, axis=0, keepdims=True)
    dot = jax.lax.dot_general(pr, pc, (((1,), (0,)), ((), ())),
                              preferred_element_type=jnp.float32)
    d2 = (sq_r + sq_c) - 2.0 * dot
    col = lax.broadcasted_iota(jnp.int32, (KROWS, NPAD), 1)
    d2 = jnp.where(col < NPTS, d2, INF)
    ams = []
    for _ in range(NSAMP):
        m = jnp.min(d2, axis=1, keepdims=True)
        eq = d2 == m
        am = jnp.min(jnp.where(eq, col, NPAD), axis=1,
                     keepdims=True)
        ams.append(am)
        d2 = jnp.where(eq, INF, d2)
    idx_ref[...] = jnp.concatenate(ams, axis=1)


def _knn(p_pad8, pT):
    return pl.pallas_call(
        _knn_body,
        grid=(KGRID,),
        in_specs=[pl.BlockSpec((KROWS, 8), lambda i: (i, 0)),
                  pl.BlockSpec((8, NPAD), lambda i: (0, 0))],
        out_specs=pl.BlockSpec((KROWS, NSAMP), lambda i: (i, 0)),
        out_shape=jax.ShapeDtypeStruct((NPAD, NSAMP), jnp.int32),
    )(p_pad8, pT)


def _sc_gather(xk, xv, p16, idxf):
    info = plsc.get_sparse_core_info()
    nw = info.num_cores * info.num_subcores
    bpw = BPAIR // nw
    chunk = 128
    nch = bpw // chunk
    nc = info.num_cores
    mesh = plsc.VectorSubcoreMesh(core_axis_name="c", subcore_axis_name="s")

    @functools.partial(
        pl.kernel, mesh=mesh,
        out_type=(jax.ShapeDtypeStruct((BPAIR, CH), jnp.float32),
                  jax.ShapeDtypeStruct((BPAIR, CH), jnp.float32),
                  jax.ShapeDtypeStruct((BPAIR, CH), jnp.float32)),
        scratch_types=[pltpu.VMEM((chunk,), jnp.int32),
                       pltpu.VMEM((chunk, CH), jnp.float32),
                       pltpu.VMEM((chunk, CH), jnp.float32),
                       pltpu.VMEM((chunk, CH), jnp.float32),
                       pltpu.SemaphoreType.DMA,
                       pltpu.SemaphoreType.DMA,
                       pltpu.SemaphoreType.DMA],
    )
    def gk(xk_h, xv_h, p_h, idx_h, okg, ovg, opg, idx_c, kb, vb, pb, s1, s2, s3):
        wid = lax.axis_index("s") * nc + lax.axis_index("c")
        base = wid * bpw

        def body(c, carry):
            off = base + c * chunk
            pltpu.sync_copy(idx_h.at[pl.ds(off, chunk)], idx_c)
            c1 = pltpu.async_copy(xk_h.at[idx_c], kb, s1)
            c2 = pltpu.async_copy(xv_h.at[idx_c], vb, s2)
            c3 = pltpu.async_copy(p_h.at[idx_c], pb, s3)
            c1.wait()
            c2.wait()
            c3.wait()
            pltpu.sync_copy(kb, okg.at[pl.ds(off, chunk)])
            pltpu.sync_copy(vb, ovg.at[pl.ds(off, chunk)])
            pltpu.sync_copy(pb, opg.at[pl.ds(off, chunk)])
            return carry

        lax.fori_loop(0, nch, body, 0)

    return gk(xk, xv, p16, idxf)


def _valid_mask(pid):
    row = lax.broadcasted_iota(jnp.int32, (RBLK, 1), 0)
    return jnp.where(pid * RBLK + row < NPAIR, 1.0, 0.0).astype(jnp.float32)


def _acc_init(pid, refs):
    @pl.when(pid == 0)
    def _():
        for r in refs:
            r[...] = jnp.zeros_like(r)


def _bn_scale(s1_ref, s2_ref, g, beta):
    mu = s1_ref[0:1, :] / NPAIR
    var = s2_ref[0:1, :] / NPAIR - mu * mu
    inv = g / jnp.sqrt(var + EPS)
    return mu, inv, beta


def _p_r(pg, wp1_ref, bp1_ref, mu_h, inv_h, betap, wp2_ref, bp2_ref):
    h = jnp.dot(pg, wp1_ref[...], preferred_element_type=jnp.float32) + bp1_ref[...]
    hbn = jnp.maximum((h - mu_h) * inv_h + betap, 0.0)
    return jnp.dot(hbn, wp2_ref[...], preferred_element_type=jnp.float32) + bp2_ref[...]


def _w0(kg_ref, xq_ref, pr):
    xq = xq_ref[...]
    xqe = jnp.reshape(jnp.broadcast_to(xq[:, None, :], (PBLK, NSAMP, CH)),
                      (RBLK, CH))
    return (kg_ref[...] - xqe) + pr


def _hstat_body(pg_ref, wp1_ref, bp1_ref, s1_ref, s2_ref):
    pid = pl.program_id(0)
    _acc_init(pid, (s1_ref, s2_ref))
    pg = pg_ref[...]
    h = jnp.dot(pg, wp1_ref[...], preferred_element_type=jnp.float32) + bp1_ref[...]
    v = _valid_mask(pid)
    s1_ref[...] += jnp.broadcast_to(jnp.sum(h * v, axis=0, keepdims=True), (8, WD))
    s2_ref[...] += jnp.broadcast_to(jnp.sum(h * h * v, axis=0, keepdims=True), (8, WD))


def _hstats(pg, Wp1p, bp1p):
    stat = pl.BlockSpec((8, WD), lambda i: (0, 0))
    return pl.pallas_call(
        _hstat_body,
        grid=(NBLK,),
        in_specs=[pl.BlockSpec((RBLK, CH), lambda i: (i, 0)),
                  pl.BlockSpec((CH, WD), lambda i: (0, 0)),
                  pl.BlockSpec((1, WD), lambda i: (0, 0))],
        out_specs=[stat, stat],
        out_shape=[jax.ShapeDtypeStruct((8, WD), jnp.float32)] * 2,
    )(pg, Wp1p, bp1p)


def _w0stat_body(kg_ref, pg_ref, xq_ref, hs1_ref, hs2_ref, wp1_ref, bp1_ref,
                 gp_ref, bep_ref, wp2_ref, bp2_ref, s1_ref, s2_ref):
    pid = pl.program_id(0)
    _acc_init(pid, (s1_ref, s2_ref))
    mu_h, inv_h, betap = _bn_scale(hs1_ref, hs2_ref, gp_ref[...], bep_ref[...])
    pr = _p_r(pg_ref[...], wp1_ref, bp1_ref, mu_h, inv_h, betap, wp2_ref, bp2_ref)
    w0 = _w0(kg_ref, xq_ref, pr)
    v = _valid_mask(pid)
    s1_ref[...] += jnp.broadcast_to(jnp.sum(w0 * v, axis=0, keepdims=True), (8, CH))
    s2_ref[...] += jnp.broadcast_to(jnp.sum(w0 * w0 * v, axis=0, keepdims=True), (8, CH))


def _w0stats(kg, pg, xq, hs1, hs2, Wp1p, bp1p, gPp, betaPp, Wp2p, bp2r):
    stat = pl.BlockSpec((8, CH), lambda i: (0, 0))
    stat16 = pl.BlockSpec((8, WD), lambda i: (0, 0))
    return pl.pallas_call(
        _w0stat_body,
        grid=(NBLK,),
        in_specs=[pl.BlockSpec((RBLK, CH), lambda i: (i, 0)),
                  pl.BlockSpec((RBLK, CH), lambda i: (i, 0)),
                  pl.BlockSpec((PBLK, CH), lambda i: (i, 0)),
                  stat16, stat16,
                  pl.BlockSpec((CH, WD), lambda i: (0, 0)),
                  pl.BlockSpec((1, WD), lambda i: (0, 0)),
                  pl.BlockSpec((1, WD), lambda i: (0, 0)),
                  pl.BlockSpec((1, WD), lambda i: (0, 0)),
                  pl.BlockSpec((WD, CH), lambda i: (0, 0)),
                  pl.BlockSpec((1, CH), lambda i: (0, 0))],
        out_specs=[stat, stat],
        out_shape=[jax.ShapeDtypeStruct((8, CH), jnp.float32)] * 2,
    )(kg, pg, xq, hs1, hs2, Wp1p, bp1p, gPp, betaPp, Wp2p, bp2r)


def _w1_body(kg_ref, pg_ref, xq_ref, hs1_ref, hs2_ref, ws1_ref, ws2_ref,
             wp1_ref, bp1_ref, gp_ref, bep_ref, wp2_ref, bp2_ref,
             g1_ref, be1_ref, w1_ref, b1_ref, w1out_ref, s1_ref, s2_ref):
    pid = pl.program_id(0)
    _acc_init(pid, (s1_ref, s2_ref))
    mu_h, inv_h, betap = _bn_scale(hs1_ref, hs2_ref, gp_ref[...], bep_ref[...])
    pr = _p_r(pg_ref[...], wp1_ref, bp1_ref, mu_h, inv_h, betap, wp2_ref, bp2_ref)
    w0 = _w0(kg_ref, xq_ref, pr)
    mu1, inv1, beta1 = _bn_scale(ws1_ref, ws2_ref, g1_ref[...], be1_ref[...])
    w0bn = jnp.maximum((w0 - mu1) * inv1 + beta1, 0.0)
    w1 = jnp.dot(w0bn, w1_ref[...], preferred_element_type=jnp.float32) + b1_ref[...]
    w1out_ref[...] = w1
    v = _valid_mask(pid)
    s1_ref[...] += jnp.broadcast_to(jnp.sum(w1 * v, axis=0, keepdims=True), (8, WD))
    s2_ref[...] += jnp.broadcast_to(jnp.sum(w1 * w1 * v, axis=0, keepdims=True), (8, WD))


def _w1pass(kg, pg, xq, hs1, hs2, ws1, ws2, Wp1p, bp1p, gPp, betaPp, Wp2p,
            bp2r, g1r, beta1r, W1, b1r):
    stat = pl.BlockSpec((8, WD), lambda i: (0, 0))
    statc = pl.BlockSpec((8, CH), lambda i: (0, 0))
    return pl.pallas_call(
        _w1_body,
        grid=(NBLK,),
        in_specs=[pl.BlockSpec((RBLK, CH), lambda i: (i, 0)),
                  pl.BlockSpec((RBLK, CH), lambda i: (i, 0)),
                  pl.BlockSpec((PBLK, CH), lambda i: (i, 0)),
                  stat, stat, statc, statc,
                  pl.BlockSpec((CH, WD), lambda i: (0, 0)),
                  pl.BlockSpec((1, WD), lambda i: (0, 0)),
                  pl.BlockSpec((1, WD), lambda i: (0, 0)),
                  pl.BlockSpec((1, WD), lambda i: (0, 0)),
                  pl.BlockSpec((WD, CH), lambda i: (0, 0)),
                  pl.BlockSpec((1, CH), lambda i: (0, 0)),
                  pl.BlockSpec((1, CH), lambda i: (0, 0)),
                  pl.BlockSpec((1, CH), lambda i: (0, 0)),
                  pl.BlockSpec((CH, WD), lambda i: (0, 0)),
                  pl.BlockSpec((1, WD), lambda i: (0, 0))],
        out_specs=[pl.BlockSpec((RBLK, WD), lambda i: (i, 0)), stat, stat],
        out_shape=[jax.ShapeDtypeStruct((BPAIR, WD), jnp.float32),
                   jax.ShapeDtypeStruct((8, WD), jnp.float32),
                   jax.ShapeDtypeStruct((8, WD), jnp.float32)],
    )(kg, pg, xq, hs1, hs2, ws1, ws2, Wp1p, bp1p, gPp, betaPp, Wp2p, bp2r,
      g1r, beta1r, W1, b1r)


def _final_body(w1_ref, vg_ref, pg_ref, w1s1_ref, w1s2_ref, hs1_ref, hs2_ref,
                wp1_ref, bp1_ref, gp_ref, bep_ref, wp2_ref, bp2_ref,
                g2_ref, be2_ref, w2_ref, b2_ref, tt_ref, out_ref):
    mu_h, inv_h, betap = _bn_scale(hs1_ref, hs2_ref, gp_ref[...], bep_ref[...])
    pr = _p_r(pg_ref[...], wp1_ref, bp1_ref, mu_h, inv_h, betap, wp2_ref, bp2_ref)
    mu2, inv2, beta2 = _bn_scale(w1s1_ref, w1s2_ref, g2_ref[...], be2_ref[...])
    w1bn = jnp.maximum((w1_ref[...] - mu2) * inv2 + beta2, 0.0)
    w2 = jnp.dot(w1bn, w2_ref[...], preferred_element_type=jnp.float32) + b2_ref[...]
    w2g = jnp.reshape(w2, (PBLK, NSAMP, WD))
    m = jnp.max(w2g, axis=1, keepdims=True)
    e = jnp.exp(w2g - m)
    s = jnp.sum(e, axis=1, keepdims=True)
    wsoft = jnp.reshape(e / s, (RBLK, WD))
    wfull = jnp.dot(wsoft, tt_ref[...], preferred_element_type=jnp.float32)
    vv = vg_ref[...] + pr
    prod = jnp.reshape(vv * wfull, (PBLK, NSAMP, CH))
    out_ref[...] = jnp.sum(prod, axis=1)


def _finalpass(w1, vg, pg, w1s1, w1s2, hs1, hs2, Wp1p, bp1p, gPp, betaPp,
               Wp2p, bp2r, g2r, beta2r, W2p, b2r, Ttile):
    stat = pl.BlockSpec((8, WD), lambda i: (0, 0))
    return pl.pallas_call(
        _final_body,
        grid=(NBLK,),
        in_specs=[pl.BlockSpec((RBLK, WD), lambda i: (i, 0)),
                  pl.BlockSpec((RBLK, CH), lambda i: (i, 0)),
                  pl.BlockSpec((RBLK, CH), lambda i: (i, 0)),
                  stat, stat, stat, stat,
                  pl.BlockSpec((CH, WD), lambda i: (0, 0)),
                  pl.BlockSpec((1, WD), lambda i: (0, 0)),
                  pl.BlockSpec((1, WD), lambda i: (0, 0)),
                  pl.BlockSpec((1, WD), lambda i: (0, 0)),
                  pl.BlockSpec((WD, CH), lambda i: (0, 0)),
                  pl.BlockSpec((1, CH), lambda i: (0, 0)),
                  pl.BlockSpec((1, WD), lambda i: (0, 0)),
                  pl.BlockSpec((1, WD), lambda i: (0, 0)),
                  pl.BlockSpec((WD, WD), lambda i: (0, 0)),
                  pl.BlockSpec((1, WD), lambda i: (0, 0)),
                  pl.BlockSpec((WD, CH), lambda i: (0, 0))],
        out_specs=pl.BlockSpec((PBLK, CH), lambda i: (i, 0)),
        out_shape=jax.ShapeDtypeStruct((NPAD, CH), jnp.float32),
    )(w1, vg, pg, w1s1, w1s2, hs1, hs2, Wp1p, bp1p, gPp, betaPp, Wp2p, bp2r,
      g2r, beta2r, W2p, b2r, Ttile)


def kernel(p, x, o, Wq, bq, Wk, bk, Wv, bv, Wp1, bp1, gP, betaP, Wp2, bp2,
           g1, beta1, W1, b1, g2, beta2, W2, b2):
    f32 = jnp.float32
    p = p.astype(f32)
    pad_n = NPAD - NPTS
    p_pad8 = jnp.pad(p, ((0, pad_n), (0, 5)))
    pT = p_pad8.T
    p128 = jnp.pad(p, ((0, pad_n), (0, CH - 3)))
    x_pad = jnp.pad(x, ((0, pad_n), (0, 0)))

    Wp1p = jnp.pad(Wp1, ((0, CH - 3), (0, WD - 3)))
    bp1p = jnp.pad(bp1, (0, WD - 3)).reshape(1, WD)
    gPp = jnp.pad(gP, (0, WD - 3)).reshape(1, WD)
    betaPp = jnp.pad(betaP, (0, WD - 3)).reshape(1, WD)
    Wp2p = jnp.pad(Wp2, ((0, WD - 3), (0, 0)))
    bp2r = bp2.reshape(1, CH)
    g1r = g1.reshape(1, CH)
    beta1r = beta1.reshape(1, CH)
    b1r = b1.reshape(1, WD)
    g2r = g2.reshape(1, WD)
    beta2r = beta2.reshape(1, WD)
    b2r = b2.reshape(1, WD)
    Ttile = jnp.tile(jnp.eye(WD, dtype=f32), (1, CH // WD))

    xq, xk, xv = _projections(x_pad, Wq, bq, Wk, bk, Wv, bv)
    idx = _knn(p_pad8, pT)
    idxf = idx.reshape(BPAIR)
    kg, vg, pg = _sc_gather(xk, xv, p128, idxf)

    hs1, hs2 = _hstats(pg, Wp1p, bp1p)
    ws1, ws2 = _w0stats(kg, pg, xq, hs1, hs2, Wp1p, bp1p, gPp, betaPp, Wp2p, bp2r)
    w1, w1s1, w1s2 = _w1pass(kg, pg, xq, hs1, hs2, ws1, ws2, Wp1p, bp1p, gPp,
                             betaPp, Wp2p, bp2r, g1r, beta1r, W1, b1r)
    out = _finalpass(w1, vg, pg, w1s1, w1s2, hs1, hs2, Wp1p, bp1p, gPp, betaPp,
                     Wp2p, bp2r, g2r, beta2r, W2, b2r, Ttile)
    return out[:NPTS]
